# baseline (speedup 1.0000x reference)
import functools

import jax
import jax.numpy as jnp
from jax import lax
from jax.experimental import pallas as pl
from jax.experimental.pallas import tpu as pltpu
from jax.experimental.pallas import tpu_sc as plsc

L = 16
NUM_BITS = 8
PERM_SIZE = 256
META_SIZE = 3 * L + PERM_SIZE
BLOCK_ROWS = 16
NBUF = 8
LOOKAHEAD = 2


def _body(num_workers, n_rows, n_cols, x_hbm, meta_hbm, out_hbm,
          meta_v, *rest):
    wid = lax.axis_index("s") * 2 + lax.axis_index("c")
    rows_per_worker = n_rows // num_workers
    n_blocks = rows_per_worker // BLOCK_ROWS
    bufs = rest[:NBUF]
    in_sems = rest[NBUF:2 * NBUF]
    out_sems = rest[2 * NBUF:3 * NBUF]

    base = wid * rows_per_worker

    def in_slice(b):
        return x_hbm.at[pl.ds(base + b * BLOCK_ROWS, BLOCK_ROWS)]

    def out_slice(b):
        return out_hbm.at[pl.ds(base + b * BLOCK_ROWS, BLOCK_ROWS)]

    pltpu.sync_copy(meta_hbm, meta_v)

    iota = lax.iota(jnp.int32, L)
    col_b = [plsc.load_gather(meta_v, [jnp.full((L,), L + j, jnp.int32)])
             for j in range(NUM_BITS)]
    bit_b = [plsc.load_gather(meta_v, [jnp.full((L,), 2 * L + j, jnp.int32)])
             for j in range(NUM_BITS)]
    perm_off = jnp.full((L,), 3 * L, jnp.int32)

    def compute(slot):
        buf = bufs[slot]
        for g in range(BLOCK_ROWS // L):
            rows = jnp.full((L,), g * L, jnp.int32) + iota
            v = jnp.zeros((L,), jnp.int32)
            for j in range(NUM_BITS):
                flag = plsc.load_gather(buf, [rows, col_b[j]])
                v = v + flag * bit_b[j]
            pv = plsc.load_gather(meta_v, [perm_off + v])
            for j in range(NUM_BITS):
                nb = jnp.where((pv & bit_b[j]) != 0,
                               jnp.full((L,), 1, jnp.int32),
                               jnp.full((L,), 0, jnp.int32))
                plsc.store_scatter(buf, [rows, col_b[j]], nb)

    for p in range(LOOKAHEAD):
        pltpu.async_copy(in_slice(p), bufs[p], in_sems[p])

    slack = NBUF - LOOKAHEAD

    @pl.loop(0, n_blocks, step=NBUF)
    def _(step):
        for sub in range(NBUF):
            b = step + sub
            slot = sub
            nxt = (sub + LOOKAHEAD) % NBUF
            pltpu.make_async_copy(in_slice(b), bufs[slot],
                                  in_sems[slot]).wait()
            compute(slot)
            pltpu.async_copy(bufs[slot], out_slice(b), out_sems[slot])

            @pl.when(b + LOOKAHEAD < n_blocks)
            def _():
                @pl.when(b >= slack)
                def _():
                    pltpu.make_async_copy(bufs[nxt], out_slice(b - slack),
                                          out_sems[nxt]).wait()
                pltpu.async_copy(in_slice(b + LOOKAHEAD), bufs[nxt],
                                 in_sems[nxt])

    for b in range(n_blocks - NBUF, n_blocks):
        slot = b % NBUF
        pltpu.make_async_copy(bufs[slot], out_slice(b),
                              out_sems[slot]).wait()


def _build(n_rows, n_cols, num_workers, interpret=False):
    mesh = plsc.VectorSubcoreMesh(core_axis_name="c", subcore_axis_name="s")
    return pl.kernel(
        functools.partial(_body, num_workers, n_rows, n_cols),
        mesh=mesh,
        out_type=jax.ShapeDtypeStruct((n_rows, n_cols), jnp.int32),
        compiler_params=pltpu.CompilerParams(needs_layout_passes=False),
        interpret=interpret,
        scratch_types=(
            [pltpu.VMEM((META_SIZE,), jnp.int32)]
            + [pltpu.VMEM((BLOCK_ROWS, n_cols), jnp.int32)
               for _ in range(NBUF)]
            + [pltpu.SemaphoreType.DMA for _ in range(2 * NBUF)]
        ),
    )


def kernel(x, indices, perm, bits):
    n_rows, n_cols = x.shape
    info = plsc.get_sparse_core_info()
    num_workers = info.num_cores * info.num_subcores

    pad = jnp.zeros((L - NUM_BITS,), jnp.int32)
    meta = jnp.concatenate([jnp.zeros((L,), jnp.int32),
                            indices.astype(jnp.int32), pad,
                            bits.astype(jnp.int32), pad,
                            perm.astype(jnp.int32)])
    f = _build(n_rows, n_cols, num_workers)
    return f(x, meta)


# speedup vs baseline: 1.1057x; 1.1057x over previous
import functools

import jax
import jax.numpy as jnp
from jax import lax
from jax.experimental import pallas as pl
from jax.experimental.pallas import tpu as pltpu
from jax.experimental.pallas import tpu_sc as plsc

L = 16
NUM_BITS = 8
PERM_SIZE = 256
META_SIZE = 3 * L + PERM_SIZE
BLOCK_ROWS = 16
NBUF = 8
LOOKAHEAD = 6


def _body(num_workers, n_rows, n_cols, x_hbm, meta_hbm, out_hbm,
          meta_v, *rest):
    wid = lax.axis_index("s") * 2 + lax.axis_index("c")
    rows_per_worker = n_rows // num_workers
    n_blocks = rows_per_worker // BLOCK_ROWS
    bufs = rest[:NBUF]
    in_sems = rest[NBUF:2 * NBUF]
    out_sems = rest[2 * NBUF:3 * NBUF]

    base = wid * rows_per_worker

    def in_slice(b):
        return x_hbm.at[pl.ds(base + b * BLOCK_ROWS, BLOCK_ROWS)]

    def out_slice(b):
        return out_hbm.at[pl.ds(base + b * BLOCK_ROWS, BLOCK_ROWS)]

    pltpu.sync_copy(meta_hbm, meta_v)

    iota = lax.iota(jnp.int32, L)
    col_b = [plsc.load_gather(meta_v, [jnp.full((L,), L + j, jnp.int32)])
             for j in range(NUM_BITS)]
    bit_b = [plsc.load_gather(meta_v, [jnp.full((L,), 2 * L + j, jnp.int32)])
             for j in range(NUM_BITS)]
    perm_off = jnp.full((L,), 3 * L, jnp.int32)

    def compute(slot):
        buf = bufs[slot]
        for g in range(BLOCK_ROWS // L):
            rows = jnp.full((L,), g * L, jnp.int32) + iota
            v = jnp.zeros((L,), jnp.int32)
            for j in range(NUM_BITS):
                flag = plsc.load_gather(buf, [rows, col_b[j]])
                v = v + flag * bit_b[j]
            pv = plsc.load_gather(meta_v, [perm_off + v])
            for j in range(NUM_BITS):
                nb = jnp.where((pv & bit_b[j]) != 0,
                               jnp.full((L,), 1, jnp.int32),
                               jnp.full((L,), 0, jnp.int32))
                plsc.store_scatter(buf, [rows, col_b[j]], nb)

    for p in range(LOOKAHEAD):
        pltpu.async_copy(in_slice(p), bufs[p], in_sems[p])

    slack = NBUF - LOOKAHEAD

    @pl.loop(0, n_blocks, step=NBUF)
    def _(step):
        for sub in range(NBUF):
            b = step + sub
            slot = sub
            nxt = (sub + LOOKAHEAD) % NBUF
            pltpu.make_async_copy(in_slice(b), bufs[slot],
                                  in_sems[slot]).wait()
            compute(slot)
            pltpu.async_copy(bufs[slot], out_slice(b), out_sems[slot])

            @pl.when(b + LOOKAHEAD < n_blocks)
            def _():
                @pl.when(b >= slack)
                def _():
                    pltpu.make_async_copy(bufs[nxt], out_slice(b - slack),
                                          out_sems[nxt]).wait()
                pltpu.async_copy(in_slice(b + LOOKAHEAD), bufs[nxt],
                                 in_sems[nxt])

    for b in range(n_blocks - NBUF, n_blocks):
        slot = b % NBUF
        pltpu.make_async_copy(bufs[slot], out_slice(b),
                              out_sems[slot]).wait()


def _build(n_rows, n_cols, num_workers, interpret=False):
    mesh = plsc.VectorSubcoreMesh(core_axis_name="c", subcore_axis_name="s")
    return pl.kernel(
        functools.partial(_body, num_workers, n_rows, n_cols),
        mesh=mesh,
        out_type=jax.ShapeDtypeStruct((n_rows, n_cols), jnp.int32),
        compiler_params=pltpu.CompilerParams(needs_layout_passes=False),
        interpret=interpret,
        scratch_types=(
            [pltpu.VMEM((META_SIZE,), jnp.int32)]
            + [pltpu.VMEM((BLOCK_ROWS, n_cols), jnp.int32)
               for _ in range(NBUF)]
            + [pltpu.SemaphoreType.DMA for _ in range(2 * NBUF)]
        ),
    )


def kernel(x, indices, perm, bits):
    n_rows, n_cols = x.shape
    info = plsc.get_sparse_core_info()
    num_workers = info.num_cores * info.num_subcores

    pad = jnp.zeros((L - NUM_BITS,), jnp.int32)
    meta = jnp.concatenate([jnp.zeros((L,), jnp.int32),
                            indices.astype(jnp.int32), pad,
                            bits.astype(jnp.int32), pad,
                            perm.astype(jnp.int32)])
    f = _build(n_rows, n_cols, num_workers)
    return f(x, meta)


# speedup vs baseline: 1.1058x; 1.0001x over previous
import functools

import jax
import jax.numpy as jnp
from jax import lax
from jax.experimental import pallas as pl
from jax.experimental.pallas import tpu as pltpu
from jax.experimental.pallas import tpu_sc as plsc

L = 16
NUM_BITS = 8
PERM_SIZE = 256
META_SIZE = 3 * L + PERM_SIZE
BLOCK_ROWS = 16
NBUF = 8
LOOKAHEAD = 6


def _body(num_workers, n_rows, n_cols, x_hbm, meta_hbm, out_hbm,
          meta_v, *rest):
    wid = lax.axis_index("s") * 2 + lax.axis_index("c")
    rows_per_worker = n_rows // num_workers
    n_blocks = rows_per_worker // BLOCK_ROWS
    bufs = rest[:NBUF]
    in_sems = rest[NBUF:2 * NBUF]
    out_sems = rest[2 * NBUF:3 * NBUF]

    base = wid * rows_per_worker

    def in_slice(b):
        return x_hbm.at[pl.ds(base + b * BLOCK_ROWS, BLOCK_ROWS)]

    def out_slice(b):
        return out_hbm.at[pl.ds(base + b * BLOCK_ROWS, BLOCK_ROWS)]

    for p in range(LOOKAHEAD):
        pltpu.async_copy(in_slice(p), bufs[p], in_sems[p])

    pltpu.sync_copy(meta_hbm, meta_v)

    iota = lax.iota(jnp.int32, L)
    col_b = [plsc.load_gather(meta_v, [jnp.full((L,), L + j, jnp.int32)])
             for j in range(NUM_BITS)]
    bit_b = [plsc.load_gather(meta_v, [jnp.full((L,), 2 * L + j, jnp.int32)])
             for j in range(NUM_BITS)]
    perm_off = jnp.full((L,), 3 * L, jnp.int32)

    def compute(slot):
        buf = bufs[slot]
        for g in range(BLOCK_ROWS // L):
            rows = jnp.full((L,), g * L, jnp.int32) + iota
            v = jnp.zeros((L,), jnp.int32)
            for j in range(NUM_BITS):
                flag = plsc.load_gather(buf, [rows, col_b[j]])
                v = v + flag * bit_b[j]
            pv = plsc.load_gather(meta_v, [perm_off + v])
            for j in range(NUM_BITS):
                nb = jnp.where((pv & bit_b[j]) != 0,
                               jnp.full((L,), 1, jnp.int32),
                               jnp.full((L,), 0, jnp.int32))
                plsc.store_scatter(buf, [rows, col_b[j]], nb)

    slack = NBUF - LOOKAHEAD

    @pl.loop(0, n_blocks, step=NBUF)
    def _(step):
        for sub in range(NBUF):
            b = step + sub
            slot = sub
            nxt = (sub + LOOKAHEAD) % NBUF
            pltpu.make_async_copy(in_slice(b), bufs[slot],
                                  in_sems[slot]).wait()
            compute(slot)
            pltpu.async_copy(bufs[slot], out_slice(b), out_sems[slot])

            @pl.when(b + LOOKAHEAD < n_blocks)
            def _():
                @pl.when(b >= slack)
                def _():
                    pltpu.make_async_copy(bufs[nxt], out_slice(b - slack),
                                          out_sems[nxt]).wait()
                pltpu.async_copy(in_slice(b + LOOKAHEAD), bufs[nxt],
                                 in_sems[nxt])

    for b in range(n_blocks - NBUF, n_blocks):
        slot = b % NBUF
        pltpu.make_async_copy(bufs[slot], out_slice(b),
                              out_sems[slot]).wait()


def _build(n_rows, n_cols, num_workers, interpret=False):
    mesh = plsc.VectorSubcoreMesh(core_axis_name="c", subcore_axis_name="s")
    return pl.kernel(
        functools.partial(_body, num_workers, n_rows, n_cols),
        mesh=mesh,
        out_type=jax.ShapeDtypeStruct((n_rows, n_cols), jnp.int32),
        compiler_params=pltpu.CompilerParams(needs_layout_passes=False),
        interpret=interpret,
        scratch_types=(
            [pltpu.VMEM((META_SIZE,), jnp.int32)]
            + [pltpu.VMEM((BLOCK_ROWS, n_cols), jnp.int32)
               for _ in range(NBUF)]
            + [pltpu.SemaphoreType.DMA for _ in range(2 * NBUF)]
        ),
    )


def kernel(x, indices, perm, bits):
    n_rows, n_cols = x.shape
    info = plsc.get_sparse_core_info()
    num_workers = info.num_cores * info.num_subcores

    pad = jnp.zeros((L - NUM_BITS,), jnp.int32)
    meta = jnp.concatenate([jnp.zeros((L,), jnp.int32),
                            indices.astype(jnp.int32), pad,
                            bits.astype(jnp.int32), pad,
                            perm.astype(jnp.int32)])
    f = _build(n_rows, n_cols, num_workers)
    return f(x, meta)
